# Initial kernel scaffold; baseline (speedup 1.0000x reference)
import functools

import jax
import jax.numpy as jnp
from jax import lax
from jax.experimental import pallas as pl
from jax.experimental.pallas import tpu as pltpu
from jax.experimental.pallas import tpu_sc as plsc

N_V = 100000
N_D = 32
BATCH = 16384
HIST = 50

NC = 2
NS = 16
NW = NC * NS

B = BATCH * HIST
B_PER_W = B // NW
CHUNK = 1600
N_CHUNKS = B_PER_W // CHUNK

_mesh = plsc.VectorSubcoreMesh(core_axis_name="c", subcore_axis_name="s")


@functools.partial(
    pl.kernel,
    out_type=jax.ShapeDtypeStruct((B, N_D), jnp.float32),
    mesh=_mesh,
    scratch_types=[
        pltpu.VMEM((CHUNK,), jnp.int32),
        pltpu.VMEM((CHUNK, N_D), jnp.float32),
        pltpu.SemaphoreType.DMA,
    ],
)
def _gather_kernel(idx_hbm, table_hbm, out_hbm, idx_v, rows_v, sem):
    wid = lax.axis_index("s") * NC + lax.axis_index("c")
    base = wid * B_PER_W

    @pl.loop(0, N_CHUNKS)
    def _chunk(i):
        off = base + i * CHUNK
        pltpu.sync_copy(idx_hbm.at[pl.ds(off, CHUNK)], idx_v)
        pltpu.async_copy(table_hbm.at[idx_v], rows_v, sem).wait()
        pltpu.sync_copy(rows_v, out_hbm.at[pl.ds(off, CHUNK)])


def kernel(input_, weight):
    idx = jnp.reshape(input_, (B,)).astype(jnp.int32)
    out = _gather_kernel(idx, weight)
    return jnp.reshape(out, (BATCH, HIST, N_D))


# speedup vs baseline: 2.9775x; 2.9775x over previous
import functools

import jax
import jax.numpy as jnp
from jax import lax
from jax.experimental import pallas as pl
from jax.experimental.pallas import tpu as pltpu
from jax.experimental.pallas import tpu_sc as plsc

N_V = 100000
N_D = 32
BATCH = 16384
HIST = 50

NC = 2
NS = 16
NW = NC * NS

B = BATCH * HIST
B_PER_W = B // NW
CHUNK = 1600
N_CHUNKS = B_PER_W // CHUNK

_mesh = plsc.VectorSubcoreMesh(core_axis_name="c", subcore_axis_name="s")


@functools.partial(
    pl.kernel,
    out_type=jax.ShapeDtypeStruct((B, N_D), jnp.float32),
    mesh=_mesh,
    scratch_types=[
        pltpu.VMEM((CHUNK,), jnp.int32),
        pltpu.VMEM((CHUNK, N_D), jnp.float32),
        pltpu.SemaphoreType.DMA,
    ],
    compiler_params=pltpu.CompilerParams(use_tc_tiling_on_sc=False),
)
def _gather_kernel(idx_hbm, table_hbm, out_hbm, idx_v, rows_v, sem):
    wid = lax.axis_index("s") * NC + lax.axis_index("c")
    base = wid * B_PER_W

    @pl.loop(0, N_CHUNKS)
    def _chunk(i):
        off = base + i * CHUNK
        pltpu.sync_copy(idx_hbm.at[pl.ds(off, CHUNK)], idx_v)
        pltpu.async_copy(table_hbm.at[idx_v], rows_v, sem).wait()
        pltpu.sync_copy(rows_v, out_hbm.at[pl.ds(off, CHUNK)])


def kernel(input_, weight):
    idx = jnp.reshape(input_, (B,)).astype(jnp.int32)
    out = _gather_kernel(idx, weight)
    return jnp.reshape(out, (BATCH, HIST, N_D))


# speedup vs baseline: 3.0154x; 1.0127x over previous
import functools

import jax
import jax.numpy as jnp
from jax import lax
from jax.experimental import pallas as pl
from jax.experimental.pallas import tpu as pltpu
from jax.experimental.pallas import tpu_sc as plsc

N_V = 100000
N_D = 32
BATCH = 16384
HIST = 50

NC = 2
NS = 16
NW = NC * NS

B = BATCH * HIST
B_PER_W = B // NW
CHUNK = 800
N_CHUNKS = B_PER_W // CHUNK
NSLOT = 4

_mesh = plsc.VectorSubcoreMesh(core_axis_name="c", subcore_axis_name="s")


@functools.partial(
    pl.kernel,
    out_type=jax.ShapeDtypeStruct((B, N_D), jnp.float32),
    mesh=_mesh,
    scratch_types=(
        [pltpu.VMEM((CHUNK,), jnp.int32) for _ in range(NSLOT)]
        + [pltpu.VMEM((CHUNK, N_D), jnp.float32) for _ in range(NSLOT)]
        + [pltpu.SemaphoreType.DMA for _ in range(3 * NSLOT)]
    ),
    compiler_params=pltpu.CompilerParams(use_tc_tiling_on_sc=False),
)
def _gather_kernel(idx_hbm, table_hbm, out_hbm, *scratch):
    idx_bufs = scratch[0:NSLOT]
    row_bufs = scratch[NSLOT:2 * NSLOT]
    sem_idx = scratch[2 * NSLOT:3 * NSLOT]
    sem_g = scratch[3 * NSLOT:4 * NSLOT]
    sem_out = scratch[4 * NSLOT:5 * NSLOT]

    wid = lax.axis_index("s") * NC + lax.axis_index("c")
    base = wid * B_PER_W

    def idx_load(i, s):
        pltpu.async_copy(idx_hbm.at[pl.ds(base + i * CHUNK, CHUNK)],
                         idx_bufs[s], sem_idx[s])

    def idx_wait(s):
        pltpu.make_async_copy(idx_hbm.at[pl.ds(0, CHUNK)],
                              idx_bufs[s], sem_idx[s]).wait()

    def gather_start(s):
        pltpu.async_copy(table_hbm.at[idx_bufs[s]], row_bufs[s], sem_g[s])

    def gather_wait(s):
        pltpu.make_async_copy(table_hbm.at[idx_bufs[s]],
                              row_bufs[s], sem_g[s]).wait()

    def store_start(i, s):
        pltpu.async_copy(row_bufs[s],
                         out_hbm.at[pl.ds(base + i * CHUNK, CHUNK)],
                         sem_out[s])

    def store_wait(s):
        pltpu.make_async_copy(row_bufs[s],
                              out_hbm.at[pl.ds(0, CHUNK)], sem_out[s]).wait()

    idx_load(0, 0)
    for i in range(3):
        idx_wait(i)
        gather_start(i)
        idx_load(i + 1, i + 1)
    idx_wait(3)
    gather_start(3)
    gather_wait(0)
    store_start(0, 0)
    idx_load(4, 0)

    @pl.loop(1, N_CHUNKS // NSLOT - 1)
    def _blk(o):
        for ph in range(NSLOT):
            i = o * NSLOT + ph
            s = ph
            sg = (ph + 1) % NSLOT
            idx_wait(s)
            store_wait(s)
            gather_start(s)
            gather_wait(sg)
            store_start(i - 3, sg)
            idx_load(i + 1, sg)

    n = N_CHUNKS
    for ph in range(NSLOT):
        i = n - NSLOT + ph
        s = ph
        sg = (ph + 1) % NSLOT
        idx_wait(s)
        store_wait(s)
        gather_start(s)
        gather_wait(sg)
        store_start(i - 3, sg)
        if ph < NSLOT - 1:
            idx_load(i + 1, sg)

    for s in range(1, NSLOT):
        gather_wait(s)
        store_start(n - NSLOT + s, s)
    for s in range(NSLOT):
        store_wait(s)


def kernel(input_, weight):
    idx = jnp.reshape(input_, (B,)).astype(jnp.int32)
    out = _gather_kernel(idx, weight)
    return jnp.reshape(out, (BATCH, HIST, N_D))
